# Initial kernel scaffold; baseline (speedup 1.0000x reference)
import jax
import jax.numpy as jnp
from jax.experimental import pallas as pl


def kernel(paragraph_variable, sentence_length_list, max_no_lines, W):
    raise NotImplementedError("write your pallas kernel here")


# speedup vs baseline: 7.3859x; 7.3859x over previous
import functools

import jax
import jax.numpy as jnp
from jax import lax
from jax.experimental import pallas as pl
from jax.experimental.pallas import tpu as pltpu
from jax.experimental.pallas import tpu_sc as plsc

B = 16
T = 4096
D = 64
L = 128
NC = 2
NS = 16
RPC = B // NC
HALF = T // 2
CH = 128
NCHUNK = HALF // CH
ROWSTRIDE = L + 8


def _body(para, slen, zeros, w, out, len_v, bnd_v, mark_v, seg2d, tok_v, rows_v,
          acc_sh, sem):
    c = lax.axis_index("c")
    s = lax.axis_index("s")
    slot = lax.rem(s, RPC)
    h = s // RPC
    b = c * RPC + slot
    t0 = h * HALF

    @pl.when(h == 0)
    def _():
        pltpu.sync_copy(zeros, acc_sh.at[pl.ds(slot * ROWSTRIDE, ROWSTRIDE)])

    pltpu.sync_copy(slen.at[b], len_v)
    pltpu.sync_copy(para.at[b, pl.ds(t0, HALF)], tok_v)

    carry = jnp.int32(0)
    for k in range(L // 16):
        v = len_v[pl.ds(k * 16, 16)]
        bnd_v[pl.ds(k * 16, 16)] = plsc.cumsum(v) + carry
        carry = carry + jnp.sum(v)

    off = jnp.int32(0)
    for k in range(L // 16):
        bv = bnd_v[pl.ds(k * 16, 16)]
        off = off + jnp.sum((bv <= t0 - 1).astype(jnp.int32))

    zero16 = jnp.zeros((16,), jnp.int32)
    for k in range(HALF // 16):
        mark_v[pl.ds(k * 16, 16)] = zero16

    lane = lax.iota(jnp.int32, 16)
    shift_idx = jnp.minimum(lane + 1, 15)
    for k in range(L // 16):
        v = bnd_v[pl.ds(k * 16, 16)]
        nxt = v.at[shift_idx].get(mode="promise_in_bounds")
        if k < L // 16 - 1:
            nv = bnd_v[pl.ds((k + 1) * 16, 16)]
            nxt = jnp.where(lane == 15, nv[0], nxt)
        else:
            nxt = jnp.where(lane == 15, jnp.int32(0x7FFFFFFF), nxt)
        rel = v - t0
        m = (v != nxt) & (rel >= 0) & (rel < HALF)
        relc = jnp.clip(rel, 0, HALF - 1)
        plsc.store_scatter(mark_v, [relc], lane + (16 * k + 1), mask=m)

    carry2 = off
    base = slot * ROWSTRIDE
    for k in range(HALF // 16):
        v = mark_v[pl.ds(k * 16, 16)]
        cm = jnp.maximum(plsc.cummax(v), carry2)
        seg2d[k // 8, pl.ds((k % 8) * 16, 16)] = cm + base
        carry2 = jnp.max(cm)

    plsc.subcore_barrier()

    for ch in range(NCHUNK):
        idx = tok_v.at[pl.ds(ch * CH, CH)]
        pltpu.async_copy(w.at[idx], rows_v, sem).wait()
        pltpu.sync_copy(rows_v, acc_sh.at[seg2d.at[ch]], add=True)

    plsc.subcore_barrier()

    @pl.when(h == 0)
    def _():
        pltpu.sync_copy(acc_sh.at[pl.ds(slot * ROWSTRIDE, L)], out.at[b])


@jax.jit
def _run(para, slen, w):
    mesh = plsc.VectorSubcoreMesh(
        core_axis_name="c", subcore_axis_name="s", num_cores=NC, num_subcores=NS
    )
    zeros = jnp.zeros((ROWSTRIDE, D), jnp.float32)
    f = pl.kernel(
        _body,
        out_type=jax.ShapeDtypeStruct((B, L, D), jnp.float32),
        mesh=mesh,
        compiler_params=pltpu.CompilerParams(needs_layout_passes=False, use_tc_tiling_on_sc=False),
        scratch_types=[
            pltpu.VMEM((L,), jnp.int32),
            pltpu.VMEM((L,), jnp.int32),
            pltpu.VMEM((HALF,), jnp.int32),
            pltpu.VMEM((NCHUNK, CH), jnp.int32),
            pltpu.VMEM((HALF,), jnp.int32),
            pltpu.VMEM((CH, D), jnp.float32),
            pltpu.VMEM_SHARED((RPC * ROWSTRIDE, D), jnp.float32),
            pltpu.SemaphoreType.DMA,
        ],
    )
    return f(para, slen, zeros, w)


def kernel(paragraph_variable, sentence_length_list, max_no_lines, W):
    del max_no_lines
    para = paragraph_variable.astype(jnp.int32)
    slen = sentence_length_list.astype(jnp.int32)
    return _run(para, slen, W)


# speedup vs baseline: 7.4251x; 1.0053x over previous
import functools

import jax
import jax.numpy as jnp
from jax import lax
from jax.experimental import pallas as pl
from jax.experimental.pallas import tpu as pltpu
from jax.experimental.pallas import tpu_sc as plsc

B = 16
T = 4096
D = 64
L = 128
NC = 2
NS = 16
RPC = B // NC
HALF = T // 2
CH = 128
NCHUNK = HALF // CH
NBUF = 8
ROWSTRIDE = L + 8


def _body(para, slen, zeros, w, out, len_v, bnd_v, mark_v, seg2d, tok_v, rows_b,
          acc_sh, sem_g, sem_s):
    c = lax.axis_index("c")
    s = lax.axis_index("s")
    slot = lax.rem(s, RPC)
    h = s // RPC
    b = c * RPC + slot
    t0 = h * HALF

    @pl.when(h == 0)
    def _():
        pltpu.sync_copy(zeros, acc_sh.at[pl.ds(slot * ROWSTRIDE, ROWSTRIDE)])

    pltpu.sync_copy(slen.at[b], len_v)
    pltpu.sync_copy(para.at[b, pl.ds(t0, HALF)], tok_v)

    carry = jnp.int32(0)
    for k in range(L // 16):
        v = len_v[pl.ds(k * 16, 16)]
        bnd_v[pl.ds(k * 16, 16)] = plsc.cumsum(v) + carry
        carry = carry + jnp.sum(v)

    off = jnp.int32(0)
    for k in range(L // 16):
        bv = bnd_v[pl.ds(k * 16, 16)]
        off = off + jnp.sum((bv <= t0 - 1).astype(jnp.int32))

    zero16 = jnp.zeros((16,), jnp.int32)
    for k in range(HALF // 16):
        mark_v[pl.ds(k * 16, 16)] = zero16

    lane = lax.iota(jnp.int32, 16)
    shift_idx = jnp.minimum(lane + 1, 15)
    for k in range(L // 16):
        v = bnd_v[pl.ds(k * 16, 16)]
        nxt = v.at[shift_idx].get(mode="promise_in_bounds")
        if k < L // 16 - 1:
            nv = bnd_v[pl.ds((k + 1) * 16, 16)]
            nxt = jnp.where(lane == 15, nv[0], nxt)
        else:
            nxt = jnp.where(lane == 15, jnp.int32(0x7FFFFFFF), nxt)
        rel = v - t0
        m = (v != nxt) & (rel >= 0) & (rel < HALF)
        relc = jnp.clip(rel, 0, HALF - 1)
        plsc.store_scatter(mark_v, [relc], lane + (16 * k + 1), mask=m)

    carry2 = off
    base = slot * ROWSTRIDE
    for k in range(HALF // 16):
        v = mark_v[pl.ds(k * 16, 16)]
        cm = jnp.maximum(plsc.cummax(v), carry2)
        seg2d[k // 8, pl.ds((k % 8) * 16, 16)] = cm + base
        carry2 = jnp.max(cm)

    plsc.subcore_barrier()

    def gstart(ch):
        idx = tok_v.at[pl.ds(ch * CH, CH)]
        return pltpu.async_copy(w.at[idx], rows_b.at[ch % NBUF], sem_g)

    gd = [None] * NCHUNK
    sd = [None] * NCHUNK
    for i in range(min(NBUF - 2, NCHUNK)):
        gd[i] = gstart(i)
    for ch in range(NCHUNK):
        if ch >= 2:
            sd[ch - 2].wait()
        nxt = ch + NBUF - 2
        if nxt < NCHUNK:
            gd[nxt] = gstart(nxt)
        gd[ch].wait()
        sd[ch] = pltpu.async_copy(
            rows_b.at[ch % NBUF], acc_sh.at[seg2d.at[ch]], sem_s, add=True
        )
    for ch in range(max(NCHUNK - 2, 0), NCHUNK):
        sd[ch].wait()

    plsc.subcore_barrier()

    @pl.when(h == 0)
    def _():
        pltpu.sync_copy(acc_sh.at[pl.ds(slot * ROWSTRIDE, L)], out.at[b])


@jax.jit
def _run(para, slen, w):
    mesh = plsc.VectorSubcoreMesh(
        core_axis_name="c", subcore_axis_name="s", num_cores=NC, num_subcores=NS
    )
    zeros = jnp.zeros((ROWSTRIDE, D), jnp.float32)
    f = pl.kernel(
        _body,
        out_type=jax.ShapeDtypeStruct((B, L, D), jnp.float32),
        mesh=mesh,
        compiler_params=pltpu.CompilerParams(needs_layout_passes=False, use_tc_tiling_on_sc=False),
        scratch_types=[
            pltpu.VMEM((L,), jnp.int32),
            pltpu.VMEM((L,), jnp.int32),
            pltpu.VMEM((HALF,), jnp.int32),
            pltpu.VMEM((NCHUNK, CH), jnp.int32),
            pltpu.VMEM((HALF,), jnp.int32),
            pltpu.VMEM((NBUF, CH, D), jnp.float32),
            pltpu.VMEM_SHARED((RPC * ROWSTRIDE, D), jnp.float32),
            pltpu.SemaphoreType.DMA,
            pltpu.SemaphoreType.DMA,
        ],
    )
    return f(para, slen, zeros, w)


def kernel(paragraph_variable, sentence_length_list, max_no_lines, W):
    del max_no_lines
    para = paragraph_variable.astype(jnp.int32)
    slen = sentence_length_list.astype(jnp.int32)
    return _run(para, slen, W)
